# baseline (speedup 1.0000x reference)
import jax
import jax.numpy as jnp
from jax.experimental import pallas as pl
from jax.experimental.pallas import tpu as pltpu

_BN = 512


def _vq_body(z_ref, e_ref, zq_ref, sse_ref):
    zb = z_ref[...]
    e = e_ref[...]
    bn = zb.shape[0]
    k = e.shape[0]

    z2 = jnp.sum(zb * zb, axis=1, keepdims=True)
    e2 = jnp.sum(e * e, axis=1)
    mm = jax.lax.dot_general(
        zb, e, (((1,), (1,)), ((), ())),
        preferred_element_type=jnp.float32)
    dist = (z2 + e2[None, :]) - 2.0 * mm

    minv = jnp.min(dist, axis=1, keepdims=True)
    iota = jax.lax.broadcasted_iota(jnp.int32, (bn, k), 1)
    idx = jnp.min(jnp.where(dist == minv, iota, k), axis=1, keepdims=True)

    onehot = (iota == idx).astype(jnp.float32)
    zq = jax.lax.dot_general(
        onehot, e, (((1,), (0,)), ((), ())),
        precision=jax.lax.Precision.HIGHEST,
        preferred_element_type=jnp.float32)

    d = zq - zb
    zq_ref[...] = zb + d

    @pl.when(pl.program_id(0) == 0)
    def _init():
        sse_ref[0, 0] = 0.0
    sse_ref[0, 0] += jnp.sum(d * d)


def kernel(z, embedding):
    B, C, H, W = z.shape
    K = embedding.shape[0]
    z_flat = jnp.transpose(z, (0, 2, 3, 1)).reshape(-1, C)
    N = z_flat.shape[0]

    zq_flat, sse = pl.pallas_call(
        _vq_body,
        grid=(N // _BN,),
        in_specs=[
            pl.BlockSpec((_BN, C), lambda i: (i, 0)),
            pl.BlockSpec((K, C), lambda i: (0, 0)),
        ],
        out_specs=[
            pl.BlockSpec((_BN, C), lambda i: (i, 0)),
            pl.BlockSpec(memory_space=pltpu.SMEM),
        ],
        out_shape=[
            jax.ShapeDtypeStruct((N, C), jnp.float32),
            jax.ShapeDtypeStruct((1, 1), jnp.float32),
        ],
    )(z_flat, embedding)

    zq_st = jnp.transpose(zq_flat.reshape(B, H, W, C), (0, 3, 1, 2))
    m = sse[0, 0] / (B * C * H * W)
    loss = m + 0.25 * m
    return (zq_st, loss)


# speedup vs baseline: 1.3123x; 1.3123x over previous
import jax
import jax.numpy as jnp
from jax.experimental import pallas as pl
from jax.experimental.pallas import tpu as pltpu

_BN = 512


def _vq_body(z_ref, e_ref, zq_ref, sse_ref):
    zb = z_ref[...]
    e = e_ref[...]
    bn = zb.shape[0]
    k = e.shape[0]

    z2 = jnp.sum(zb * zb, axis=1, keepdims=True)
    e2 = jnp.sum(e * e, axis=1)
    mm = jax.lax.dot_general(
        zb, e, (((1,), (1,)), ((), ())),
        preferred_element_type=jnp.float32)
    dist = (z2 + e2[None, :]) - 2.0 * mm

    minv = jnp.min(dist, axis=1, keepdims=True)
    iota = jax.lax.broadcasted_iota(jnp.int32, (bn, k), 1)
    idx = jnp.min(jnp.where(dist == minv, iota, k), axis=1, keepdims=True)

    onehot = (iota == idx).astype(jnp.bfloat16)
    e0 = e.astype(jnp.bfloat16)
    r1 = e - e0.astype(jnp.float32)
    e1 = r1.astype(jnp.bfloat16)
    e2c = (r1 - e1.astype(jnp.float32)).astype(jnp.bfloat16)
    dn = (((1,), (0,)), ((), ()))
    p0 = jax.lax.dot_general(onehot, e0, dn, preferred_element_type=jnp.float32)
    p1 = jax.lax.dot_general(onehot, e1, dn, preferred_element_type=jnp.float32)
    p2 = jax.lax.dot_general(onehot, e2c, dn, preferred_element_type=jnp.float32)
    zq = (p0 + p1) + p2

    d = zq - zb
    zq_ref[...] = zb + d

    @pl.when(pl.program_id(0) == 0)
    def _init():
        sse_ref[0, 0] = 0.0
    sse_ref[0, 0] += jnp.sum(d * d)


def kernel(z, embedding):
    B, C, H, W = z.shape
    K = embedding.shape[0]
    z_flat = jnp.transpose(z, (0, 2, 3, 1)).reshape(-1, C)
    N = z_flat.shape[0]

    zq_flat, sse = pl.pallas_call(
        _vq_body,
        grid=(N // _BN,),
        in_specs=[
            pl.BlockSpec((_BN, C), lambda i: (i, 0)),
            pl.BlockSpec((K, C), lambda i: (0, 0)),
        ],
        out_specs=[
            pl.BlockSpec((_BN, C), lambda i: (i, 0)),
            pl.BlockSpec(memory_space=pltpu.SMEM),
        ],
        out_shape=[
            jax.ShapeDtypeStruct((N, C), jnp.float32),
            jax.ShapeDtypeStruct((1, 1), jnp.float32),
        ],
    )(z_flat, embedding)

    zq_st = jnp.transpose(zq_flat.reshape(B, H, W, C), (0, 3, 1, 2))
    m = sse[0, 0] / (B * C * H * W)
    loss = m + 0.25 * m
    return (zq_st, loss)


# speedup vs baseline: 1.6667x; 1.2701x over previous
import jax
import jax.numpy as jnp
from jax.experimental import pallas as pl
from jax.experimental.pallas import tpu as pltpu


def _vq_body(z_ref, e_ref, e0_ref, e1_ref, e2c_ref, esq_ref, zq_ref, sse_ref):
    zc = z_ref[0]
    e = e_ref[...]
    k = e.shape[0]
    p = zc.shape[1]

    z2 = jnp.sum(zc * zc, axis=0, keepdims=True)
    esq = esq_ref[...]
    mm = jax.lax.dot_general(
        e, zc, (((1,), (0,)), ((), ())),
        preferred_element_type=jnp.float32)
    dist = (z2 + esq) - 2.0 * mm

    minv = jnp.min(dist, axis=0, keepdims=True)
    iota = jax.lax.broadcasted_iota(jnp.int32, (k, p), 0)
    idx = jnp.min(jnp.where(dist == minv, iota, k), axis=0, keepdims=True)

    onehot = (iota == idx).astype(jnp.bfloat16)
    dn = (((0,), (0,)), ((), ()))
    p0 = jax.lax.dot_general(e0_ref[...], onehot, dn,
                             preferred_element_type=jnp.float32)
    p1 = jax.lax.dot_general(e1_ref[...], onehot, dn,
                             preferred_element_type=jnp.float32)
    p2 = jax.lax.dot_general(e2c_ref[...], onehot, dn,
                             preferred_element_type=jnp.float32)
    zq = (p0 + p1) + p2

    d = zq - zc
    zq_ref[0] = zc + d

    @pl.when(pl.program_id(0) == 0)
    def _init():
        sse_ref[0, 0] = 0.0
    sse_ref[0, 0] += jnp.sum(d * d)


def kernel(z, embedding):
    B, C, H, W = z.shape
    K = embedding.shape[0]
    P = H * W
    zv = z.reshape(B, C, P)

    esq = jnp.sum(embedding ** 2, axis=1)[:, None]
    e0 = embedding.astype(jnp.bfloat16)
    r1 = embedding - e0.astype(jnp.float32)
    e1 = r1.astype(jnp.bfloat16)
    e2c = (r1 - e1.astype(jnp.float32)).astype(jnp.bfloat16)

    zq, sse = pl.pallas_call(
        _vq_body,
        grid=(B,),
        in_specs=[
            pl.BlockSpec((1, C, P), lambda i: (i, 0, 0)),
            pl.BlockSpec((K, C), lambda i: (0, 0)),
            pl.BlockSpec((K, C), lambda i: (0, 0)),
            pl.BlockSpec((K, C), lambda i: (0, 0)),
            pl.BlockSpec((K, C), lambda i: (0, 0)),
            pl.BlockSpec((K, 1), lambda i: (0, 0)),
        ],
        out_specs=[
            pl.BlockSpec((1, C, P), lambda i: (i, 0, 0)),
            pl.BlockSpec(memory_space=pltpu.SMEM),
        ],
        out_shape=[
            jax.ShapeDtypeStruct((B, C, P), jnp.float32),
            jax.ShapeDtypeStruct((1, 1), jnp.float32),
        ],
    )(zv, embedding, e0, e1, e2c, esq)

    zq_st = zq.reshape(B, C, H, W)
    m = sse[0, 0] / (B * C * H * W)
    loss = m + 0.25 * m
    return (zq_st, loss)


# speedup vs baseline: 1.8850x; 1.1310x over previous
import jax
import jax.numpy as jnp
from jax.experimental import pallas as pl
from jax.experimental.pallas import tpu as pltpu


def _vq_body(z_ref, em2_ref, ecat_ref, esq_ref, zq_ref, sse_ref):
    zc = z_ref[0]
    em2 = em2_ref[...]
    k = em2.shape[0]
    p = zc.shape[1]

    z2 = jnp.sum(zc * zc, axis=0, keepdims=True)
    mmn2 = jax.lax.dot_general(
        em2, zc, (((1,), (0,)), ((), ())),
        preferred_element_type=jnp.float32)
    dist = (z2 + esq_ref[...]) + mmn2

    minv = jnp.min(dist, axis=0, keepdims=True)
    iota = jax.lax.broadcasted_iota(jnp.int32, (k, p), 0)
    idx = jnp.min(jnp.where(dist == minv, iota, k), axis=0, keepdims=True)

    onehot = (iota == idx).astype(jnp.bfloat16)
    parts = jax.lax.dot_general(
        ecat_ref[...], onehot, (((0,), (0,)), ((), ())),
        preferred_element_type=jnp.float32)
    c = em2.shape[1]
    zq = (parts[:c] + parts[c:2 * c]) + parts[2 * c:]

    d = zq - zc
    zq_ref[0] = zc + d

    @pl.when(pl.program_id(0) == 0)
    def _init():
        sse_ref[0, 0] = 0.0
    sse_ref[0, 0] += jnp.sum(d * d)


def kernel(z, embedding):
    B, C, H, W = z.shape
    K = embedding.shape[0]
    P = H * W
    zv = z.reshape(B, C, P)

    esq = jnp.sum(embedding ** 2, axis=1)[:, None]
    em2 = -2.0 * embedding
    e0 = embedding.astype(jnp.bfloat16)
    r1 = embedding - e0.astype(jnp.float32)
    e1 = r1.astype(jnp.bfloat16)
    e2c = (r1 - e1.astype(jnp.float32)).astype(jnp.bfloat16)
    ecat = jnp.concatenate([e0, e1, e2c], axis=1)

    zq, sse = pl.pallas_call(
        _vq_body,
        grid=(B,),
        in_specs=[
            pl.BlockSpec((1, C, P), lambda i: (i, 0, 0)),
            pl.BlockSpec((K, C), lambda i: (0, 0)),
            pl.BlockSpec((K, 3 * C), lambda i: (0, 0)),
            pl.BlockSpec((K, 1), lambda i: (0, 0)),
        ],
        out_specs=[
            pl.BlockSpec((1, C, P), lambda i: (i, 0, 0)),
            pl.BlockSpec(memory_space=pltpu.SMEM),
        ],
        out_shape=[
            jax.ShapeDtypeStruct((B, C, P), jnp.float32),
            jax.ShapeDtypeStruct((1, 1), jnp.float32),
        ],
    )(zv, em2, ecat, esq)

    zq_st = zq.reshape(B, C, H, W)
    m = sse[0, 0] / (B * C * H * W)
    loss = m + 0.25 * m
    return (zq_st, loss)


# speedup vs baseline: 2.0518x; 1.0885x over previous
import jax
import jax.numpy as jnp
from jax.experimental import pallas as pl
from jax.experimental.pallas import tpu as pltpu


def _vq_body(z_ref, e_ref, zq_ref, loss_ref,
             em2_ref, ecat_ref, esq_ref, sse_ref):
    k, c = e_ref.shape
    p = z_ref.shape[2]
    nsteps = pl.num_programs(0)

    @pl.when(pl.program_id(0) == 0)
    def _prep():
        e = e_ref[...]
        esq_ref[...] = jnp.sum(e * e, axis=1, keepdims=True)
        em2_ref[...] = -2.0 * e
        e0 = e.astype(jnp.bfloat16)
        r1 = e - e0.astype(jnp.float32)
        e1 = r1.astype(jnp.bfloat16)
        e2c = (r1 - e1.astype(jnp.float32)).astype(jnp.bfloat16)
        ecat_ref[:, 0:c] = e0
        ecat_ref[:, c:2 * c] = e1
        ecat_ref[:, 2 * c:3 * c] = e2c
        sse_ref[0, 0] = 0.0

    zc = z_ref[0]
    z2 = jnp.sum(zc * zc, axis=0, keepdims=True)
    mmn2 = jax.lax.dot_general(
        em2_ref[...], zc, (((1,), (0,)), ((), ())),
        preferred_element_type=jnp.float32)
    dist = (z2 + esq_ref[...]) + mmn2

    minv = jnp.min(dist, axis=0, keepdims=True)
    iota = jax.lax.broadcasted_iota(jnp.int32, (k, p), 0)
    idx = jnp.min(jnp.where(dist == minv, iota, k), axis=0, keepdims=True)

    onehot = (iota == idx).astype(jnp.bfloat16)
    parts = jax.lax.dot_general(
        ecat_ref[...], onehot, (((0,), (0,)), ((), ())),
        preferred_element_type=jnp.float32)
    zq = (parts[:c] + parts[c:2 * c]) + parts[2 * c:]

    d = zq - zc
    zq_ref[0] = zc + d
    sse_ref[0, 0] += jnp.sum(d * d)

    @pl.when(pl.program_id(0) == nsteps - 1)
    def _fin():
        m = sse_ref[0, 0] / (nsteps * c * p)
        loss_ref[0, 0] = m + 0.25 * m


def kernel(z, embedding):
    B, C, H, W = z.shape
    K = embedding.shape[0]
    P = H * W
    zv = z.reshape(B, C, P)

    zq, loss = pl.pallas_call(
        _vq_body,
        grid=(B,),
        in_specs=[
            pl.BlockSpec((1, C, P), lambda i: (i, 0, 0)),
            pl.BlockSpec((K, C), lambda i: (0, 0)),
        ],
        out_specs=[
            pl.BlockSpec((1, C, P), lambda i: (i, 0, 0)),
            pl.BlockSpec(memory_space=pltpu.SMEM),
        ],
        out_shape=[
            jax.ShapeDtypeStruct((B, C, P), jnp.float32),
            jax.ShapeDtypeStruct((1, 1), jnp.float32),
        ],
        scratch_shapes=[
            pltpu.VMEM((K, C), jnp.float32),
            pltpu.VMEM((K, 3 * C), jnp.bfloat16),
            pltpu.VMEM((K, 1), jnp.float32),
            pltpu.SMEM((1, 1), jnp.float32),
        ],
    )(zv, embedding)

    return (zq.reshape(B, C, H, W), loss[0, 0])
